# Initial kernel scaffold; baseline (speedup 1.0000x reference)
import functools

import jax
import jax.numpy as jnp
from jax import lax
from jax.experimental import pallas as pl
from jax.experimental.pallas import tpu as pltpu
from jax.experimental.pallas import tpu_sc as plsc

S = 200
B = 4096
V = 100000
D = 128
NCLS = 5
NPAD = 8

VB = 8192
NVB = (V + VB - 1) // VB

NCHUNK = 6
CH = 688
CH_LAST = 656
NG = CH // 16
NG_LAST = CH_LAST // 16


def _proj_body(w_ref, b_ref, t_ref, y_ref):
    pid = pl.program_id(0)
    t = t_ref[...]
    rows = lax.broadcasted_iota(jnp.int32, t.shape, 0) + pid * VB
    t = jnp.where(rows == 0, 0.0, t)
    y = lax.dot_general(w_ref[...], t, (((1,), (1,)), ((), ())),
                        preferred_element_type=jnp.float32)
    y_ref[...] = (y + b_ref[...]) * (1.0 / S)


_proj_call = pl.pallas_call(
    _proj_body,
    grid=(NVB,),
    in_specs=[
        pl.BlockSpec((NPAD, D), lambda i: (0, 0)),
        pl.BlockSpec((NPAD, 1), lambda i: (0, 0)),
        pl.BlockSpec((VB, D), lambda i: (i, 0)),
    ],
    out_specs=pl.BlockSpec((NPAD, VB), lambda i: (0, i)),
    out_shape=jax.ShapeDtypeStruct((NPAD, V), jnp.float32),
)


_sc_mesh = plsc.VectorSubcoreMesh(core_axis_name="c", subcore_axis_name="s")


@functools.partial(
    pl.kernel,
    out_type=jax.ShapeDtypeStruct((NCLS, B), jnp.float32),
    mesh=_sc_mesh,
    scratch_types=[
        pltpu.VMEM((V,), jnp.float32),
        pltpu.VMEM((S, 16), jnp.int32),
        pltpu.VMEM((CH,), jnp.float32),
    ],
)
def _pool_kernel(projT, text, out, col, slab, obuf):
    wid = lax.axis_index("s") * 2 + lax.axis_index("c")

    @pl.when(wid < NCLS * NCHUNK)
    def _():
        n = wid // NCHUNK
        c = wid % NCHUNK
        b0 = c * CH
        ngroups = jnp.where(c == NCHUNK - 1, NG_LAST, NG)
        pltpu.sync_copy(projT.at[n], col)

        def per_group(g, carry):
            @pl.when(g < ngroups)
            def _():
                pltpu.sync_copy(text.at[:, pl.ds(b0 + g * 16, 16)], slab)

                def per_s(i, acc):
                    for u in range(8):
                        idx = slab[i * 8 + u, :]
                        acc = acc + plsc.load_gather(col, [idx])
                    return acc

                acc = lax.fori_loop(0, S // 8, per_s,
                                    jnp.zeros((16,), jnp.float32))
                obuf[pl.ds(g * 16, 16)] = acc
            return carry

        lax.fori_loop(0, NG, per_group, 0)

        @pl.when(c < NCHUNK - 1)
        def _():
            pltpu.sync_copy(obuf, out.at[n, pl.ds(b0, CH)])

        @pl.when(c == NCHUNK - 1)
        def _():
            pltpu.sync_copy(obuf.at[pl.ds(0, CH_LAST)],
                            out.at[n, pl.ds(b0, CH_LAST)])


def kernel(text, emb_table, W, b):
    w8 = jnp.zeros((NPAD, D), jnp.float32).at[:NCLS].set(W)
    b8 = jnp.zeros((NPAD, 1), jnp.float32).at[:NCLS, 0].set(b)
    projT = _proj_call(w8, b8, emb_table)
    outT = _pool_kernel(projT, text)
    return outT.T


# speedup vs baseline: 36.0431x; 36.0431x over previous
import functools

import jax
import jax.numpy as jnp
from jax import lax
from jax.experimental import pallas as pl
from jax.experimental.pallas import tpu as pltpu
from jax.experimental.pallas import tpu_sc as plsc

S = 200
B = 4096
V = 100000
D = 128
NCLS = 5
NPAD = 8

VB = 8192
NVB = (V + VB - 1) // VB

NCHUNK = 6
NBLK = B // 128


def _proj_body(w_ref, b_ref, t_ref, y_ref):
    pid = pl.program_id(0)
    t = t_ref[...]
    rows = lax.broadcasted_iota(jnp.int32, t.shape, 0) + pid * VB
    t = jnp.where(rows == 0, 0.0, t)
    y = lax.dot_general(w_ref[...], t, (((1,), (1,)), ((), ())),
                        preferred_element_type=jnp.float32)
    y_ref[...] = (y + b_ref[...]) * (1.0 / S)


_proj_call = pl.pallas_call(
    _proj_body,
    grid=(NVB,),
    in_specs=[
        pl.BlockSpec((NPAD, D), lambda i: (0, 0)),
        pl.BlockSpec((NPAD, 1), lambda i: (0, 0)),
        pl.BlockSpec((VB, D), lambda i: (i, 0)),
    ],
    out_specs=pl.BlockSpec((NPAD, VB), lambda i: (0, i)),
    out_shape=jax.ShapeDtypeStruct((NPAD, V), jnp.float32),
)


_sc_mesh = plsc.VectorSubcoreMesh(core_axis_name="c", subcore_axis_name="s")


@functools.partial(
    pl.kernel,
    out_type=jax.ShapeDtypeStruct((NCLS * B,), jnp.float32),
    mesh=_sc_mesh,
    compiler_params=pltpu.CompilerParams(needs_layout_passes=False),
    scratch_types=[
        pltpu.VMEM((V,), jnp.float32),
        pltpu.VMEM((S, 128), jnp.int32),
        pltpu.VMEM((128,), jnp.float32),
    ],
)
def _pool_kernel(proj_flat, text, out_flat, col, slab, obuf):
    wid = lax.axis_index("s") * 2 + lax.axis_index("c")

    @pl.when(wid < NCLS * NCHUNK)
    def _():
        n = wid // NCHUNK
        c = wid % NCHUNK
        start = 5 * c + (c >= 1).astype(jnp.int32) + (c >= 4).astype(jnp.int32)
        count = jnp.where((c == 0) | (c == 3), 6, 5)
        pltpu.sync_copy(proj_flat.at[pl.ds(pl.multiple_of(n * V, 8), V)], col)

        def per_block(j, carry):
            @pl.when(j < count)
            def _():
                blk = start + j
                col0 = pl.multiple_of(blk * 128, 128)
                pltpu.sync_copy(text.at[:, pl.ds(col0, 128)], slab)

                for sub in range(8):
                    def per_s(i, acc):
                        for u in range(8):
                            idx = slab[i * 8 + u, pl.ds(sub * 16, 16)]
                            acc = acc + plsc.load_gather(col, [idx])
                        return acc

                    acc = lax.fori_loop(0, S // 8, per_s,
                                        jnp.zeros((16,), jnp.float32))
                    obuf[pl.ds(sub * 16, 16)] = acc

                dst = pl.multiple_of(n * B + blk * 128, 8)
                pltpu.sync_copy(obuf, out_flat.at[pl.ds(dst, 128)])
            return carry

        lax.fori_loop(0, 6, per_block, 0)


def kernel(text, emb_table, W, b):
    w8 = jnp.zeros((NPAD, D), jnp.float32).at[:NCLS].set(W)
    b8 = jnp.zeros((NPAD, 1), jnp.float32).at[:NCLS, 0].set(b)
    projT = _proj_call(w8, b8, emb_table)
    proj_flat = projT[:NCLS].reshape(NCLS * V)
    out_flat = _pool_kernel(proj_flat, text)
    return out_flat.reshape(NCLS, B).T


# speedup vs baseline: 42.5578x; 1.1807x over previous
import functools

import jax
import jax.numpy as jnp
from jax import lax
from jax.experimental import pallas as pl
from jax.experimental.pallas import tpu as pltpu
from jax.experimental.pallas import tpu_sc as plsc

S = 200
B = 4096
V = 100000
D = 128
NCLS = 5

VB = 8192
NVB = (V + VB - 1) // VB

NBLK = B // 128
RH0 = 104
RH1 = 96
NT = (11, 11, 10)
MASKHI = -65536


def _proj_body(w_ref, b_ref, t_ref, y_ref):
    pid = pl.program_id(0)
    t = t_ref[...]
    rows = lax.broadcasted_iota(jnp.int32, t.shape, 0) + pid * VB
    t = jnp.where(rows == 0, 0.0, t)
    y = lax.dot_general(w_ref[...], t, (((1,), (1,)), ((), ())),
                        preferred_element_type=jnp.float32)
    y = (y + b_ref[...]) * (1.0 / S)
    u = lax.bitcast_convert_type(y.astype(jnp.bfloat16), jnp.uint16)
    u = u.astype(jnp.int32)
    r0 = (u[0:1] << 16) | u[1:2]
    r1 = (u[2:3] << 16) | u[3:4]
    r2 = (u[4:5] << 16)
    y_ref[...] = jnp.concatenate([r0, r1, r2], axis=0)


_proj_call = pl.pallas_call(
    _proj_body,
    grid=(NVB,),
    in_specs=[
        pl.BlockSpec((NCLS, D), lambda i: (0, 0)),
        pl.BlockSpec((NCLS, 1), lambda i: (0, 0)),
        pl.BlockSpec((VB, D), lambda i: (i, 0)),
    ],
    out_specs=pl.BlockSpec((3, VB), lambda i: (0, i)),
    out_shape=jax.ShapeDtypeStruct((3, V), jnp.int32),
)


_sc_mesh = plsc.VectorSubcoreMesh(core_axis_name="c", subcore_axis_name="s")


@functools.partial(
    pl.kernel,
    out_type=jax.ShapeDtypeStruct((NCLS * B,), jnp.float32),
    mesh=_sc_mesh,
    compiler_params=pltpu.CompilerParams(needs_layout_passes=False),
    scratch_types=[
        pltpu.VMEM((V,), jnp.int32),
        pltpu.VMEM((RH0, 128), jnp.int32),
        pltpu.VMEM((RH1, 128), jnp.int32),
        pltpu.VMEM((4, 2, 128), jnp.float32),
        pltpu.SemaphoreType.DMA,
        pltpu.SemaphoreType.DMA,
        pltpu.SemaphoreType.DMA,
        pltpu.SemaphoreType.DMA,
    ],
)
def _pool_kernel(pack_flat, text, out_flat, col, slabA, slabB, obuf,
                 sem_c, sem_a, sem_b, sem_o):
    wid = lax.axis_index("s") * 2 + lax.axis_index("c")
    g = (wid >= NT[0]).astype(jnp.int32) + (wid >= NT[0] + NT[1]).astype(jnp.int32)
    local = wid - jnp.where(g == 0, 0, jnp.where(g == 1, NT[0], NT[0] + NT[1]))
    nt = jnp.where(g == 0, NT[0], jnp.where(g == 1, NT[1], NT[2]))

    blks = [local + k * nt for k in range(4)]
    actives = [blk < NBLK for blk in blks]

    def slab_copy(i):
        k, h = divmod(i, 2)
        buf, sem = (slabA, sem_a) if h == 0 else (slabB, sem_b)
        r0 = 0 if h == 0 else RH0
        rows = RH0 if h == 0 else RH1
        col0 = pl.multiple_of(blks[k] * 128, 128)
        return pltpu.make_async_copy(
            text.at[pl.ds(r0, rows), pl.ds(col0, 128)], buf, sem)

    def out_copy(k, plane):
        cls = 2 * g + plane
        dst = pl.multiple_of(cls * B + blks[k] * 128, 8)
        return pltpu.make_async_copy(
            obuf.at[k, plane], out_flat.at[pl.ds(dst, 128)], sem_o)

    col_cp = pltpu.make_async_copy(
        pack_flat.at[pl.ds(pl.multiple_of(g * V, 8), V)], col, sem_c)
    col_cp.start()

    @pl.when(actives[0])
    def _():
        slab_copy(0).start()

    col_cp.wait()

    for i in range(8):
        k, h = divmod(i, 2)
        if i + 1 < 8:
            kn = (i + 1) // 2
            @pl.when(actives[kn])
            def _(i=i):
                slab_copy(i + 1).start()

        @pl.when(actives[k])
        def _(i=i, k=k, h=h):
            slab_copy(i).wait()
            buf = slabA if h == 0 else slabB
            rows = RH0 if h == 0 else RH1
            for sub in range(8):
                def body(iv, accs, buf=buf, sub=sub):
                    ah, al = accs
                    for u in range(8):
                        idx = buf[iv * 8 + u, pl.ds(sub * 16, 16)]
                        v = plsc.load_gather(col, [idx])
                        ah = ah + plsc.bitcast(v & MASKHI, jnp.float32)
                        al = al + plsc.bitcast(v << 16, jnp.float32)
                    return ah, al

                z = jnp.zeros((16,), jnp.float32)
                ah, al = lax.fori_loop(0, rows // 8, body, (z, z))
                if h == 0:
                    obuf[k, 0, pl.ds(sub * 16, 16)] = ah
                    obuf[k, 1, pl.ds(sub * 16, 16)] = al
                else:
                    plsc.addupdate(obuf.at[k, 0, pl.ds(sub * 16, 16)], ah)
                    plsc.addupdate(obuf.at[k, 1, pl.ds(sub * 16, 16)], al)
            if h == 1:
                out_copy(k, 0).start()

        if h == 1:
            @pl.when(jnp.logical_and(actives[k], g < 2))
            def _(k=k):
                out_copy(k, 1).start()

    for k in range(4):
        @pl.when(actives[k])
        def _(k=k):
            out_copy(k, 0).wait()

        @pl.when(jnp.logical_and(actives[k], g < 2))
        def _(k=k):
            out_copy(k, 1).wait()


def kernel(text, emb_table, W, b):
    b5 = b.reshape(NCLS, 1)
    packed = _proj_call(W, b5, emb_table)
    pack_flat = packed.reshape(3 * V)
    out_flat = _pool_kernel(pack_flat, text)
    return out_flat.reshape(NCLS, B).T


# speedup vs baseline: 49.1326x; 1.1545x over previous
import functools

import jax
import jax.numpy as jnp
from jax import lax
from jax.experimental import pallas as pl
from jax.experimental.pallas import tpu as pltpu
from jax.experimental.pallas import tpu_sc as plsc

S = 200
B = 4096
V = 100000
D = 128
NCLS = 5

VB = 16384
NVB = (V + VB - 1) // VB
VP = NVB * VB

NBLK = B // 128
RH0 = 104
RH1 = 96
NT = (11, 11, 10)
MASKHI = -65536


def _proj_body(w_ref, b_ref, t_ref, y0_ref, y1_ref, y2_ref):
    pid = pl.program_id(0)
    t = t_ref[...]
    rows = lax.broadcasted_iota(jnp.int32, t.shape, 0) + pid * VB
    t = jnp.where(rows == 0, 0.0, t)
    y = lax.dot_general(w_ref[...], t, (((1,), (1,)), ((), ())),
                        preferred_element_type=jnp.float32)
    y = (y + b_ref[...]) * (1.0 / S)
    u = lax.bitcast_convert_type(y.astype(jnp.bfloat16), jnp.uint16)
    u = u.astype(jnp.int32)
    y0_ref[...] = ((u[0:1] << 16) | u[1:2]).reshape(VB)
    y1_ref[...] = ((u[2:3] << 16) | u[3:4]).reshape(VB)
    y2_ref[...] = (u[4:5] << 16).reshape(VB)


_proj_call = pl.pallas_call(
    _proj_body,
    grid=(NVB,),
    in_specs=[
        pl.BlockSpec((NCLS, D), lambda i: (0, 0)),
        pl.BlockSpec((NCLS, 1), lambda i: (0, 0)),
        pl.BlockSpec((VB, D), lambda i: (i, 0)),
    ],
    out_specs=[pl.BlockSpec((VB,), lambda i: (i,)) for _ in range(3)],
    out_shape=[jax.ShapeDtypeStruct((VP,), jnp.int32) for _ in range(3)],
)


_sc_mesh = plsc.VectorSubcoreMesh(core_axis_name="c", subcore_axis_name="s")


@functools.partial(
    pl.kernel,
    out_type=jax.ShapeDtypeStruct((NCLS * B,), jnp.float32),
    mesh=_sc_mesh,
    compiler_params=pltpu.CompilerParams(needs_layout_passes=False),
    scratch_types=[
        pltpu.VMEM((V,), jnp.int32),
        pltpu.VMEM((RH0, 128), jnp.int32),
        pltpu.VMEM((RH1, 128), jnp.int32),
        pltpu.VMEM((4, 2, 128), jnp.float32),
        pltpu.SemaphoreType.DMA,
        pltpu.SemaphoreType.DMA,
        pltpu.SemaphoreType.DMA,
        pltpu.SemaphoreType.DMA,
    ],
)
def _pool_kernel(p0, p1, p2, text, out_flat, col, slabA, slabB, obuf,
                 sem_c, sem_a, sem_b, sem_o):
    wid = lax.axis_index("s") * 2 + lax.axis_index("c")
    g = (wid >= NT[0]).astype(jnp.int32) + (wid >= NT[0] + NT[1]).astype(jnp.int32)
    local = wid - jnp.where(g == 0, 0, jnp.where(g == 1, NT[0], NT[0] + NT[1]))
    nt = jnp.where(g == 0, NT[0], jnp.where(g == 1, NT[1], NT[2]))

    blks = [local + k * nt for k in range(4)]
    actives = [blk < NBLK for blk in blks]

    def slab_copy(i):
        k, h = divmod(i, 2)
        buf, sem = (slabA, sem_a) if h == 0 else (slabB, sem_b)
        r0 = 0 if h == 0 else RH0
        rows = RH0 if h == 0 else RH1
        col0 = pl.multiple_of(blks[k] * 128, 128)
        return pltpu.make_async_copy(
            text.at[pl.ds(r0, rows), pl.ds(col0, 128)], buf, sem)

    def out_copy(k, plane):
        cls = 2 * g + plane
        dst = pl.multiple_of(cls * B + blks[k] * 128, 8)
        return pltpu.make_async_copy(
            obuf.at[k, plane], out_flat.at[pl.ds(dst, 128)], sem_o)

    for gi, src in enumerate((p0, p1, p2)):
        @pl.when(g == gi)
        def _(src=src):
            pltpu.make_async_copy(src.at[pl.ds(0, V)], col, sem_c).start()

    @pl.when(actives[0])
    def _():
        slab_copy(0).start()

    pltpu.make_async_copy(p0.at[pl.ds(0, V)], col, sem_c).wait()

    for i in range(8):
        k, h = divmod(i, 2)
        if i + 1 < 8:
            kn = (i + 1) // 2
            @pl.when(actives[kn])
            def _(i=i):
                slab_copy(i + 1).start()

        @pl.when(actives[k])
        def _(i=i, k=k, h=h):
            slab_copy(i).wait()
            buf = slabA if h == 0 else slabB
            rows = RH0 if h == 0 else RH1

            def sub_body(sub, _, buf=buf, rows=rows, k=k, h=h):
                def body(iv, accs):
                    ah, al = accs
                    for u in range(8):
                        idx = buf[iv * 8 + u, pl.ds(sub * 16, 16)]
                        v = plsc.load_gather(col, [idx])
                        ah = ah + plsc.bitcast(v & MASKHI, jnp.float32)
                        al = al + plsc.bitcast(v << 16, jnp.float32)
                    return ah, al

                z = jnp.zeros((16,), jnp.float32)
                ah, al = lax.fori_loop(0, rows // 8, body, (z, z))
                if h == 0:
                    obuf[k, 0, pl.ds(sub * 16, 16)] = ah
                    obuf[k, 1, pl.ds(sub * 16, 16)] = al
                else:
                    plsc.addupdate(obuf.at[k, 0, pl.ds(sub * 16, 16)], ah)
                    plsc.addupdate(obuf.at[k, 1, pl.ds(sub * 16, 16)], al)
                return 0

            lax.fori_loop(0, 8, sub_body, 0)
            if h == 1:
                out_copy(k, 0).start()

        if h == 1:
            @pl.when(jnp.logical_and(actives[k], g < 2))
            def _(k=k):
                out_copy(k, 1).start()

    for k in range(4):
        @pl.when(actives[k])
        def _(k=k):
            out_copy(k, 0).wait()

        @pl.when(jnp.logical_and(actives[k], g < 2))
        def _(k=k):
            out_copy(k, 1).wait()


def kernel(text, emb_table, W, b):
    b5 = b.reshape(NCLS, 1)
    p0, p1, p2 = _proj_call(W, b5, emb_table)
    out_flat = _pool_kernel(p0, p1, p2, text)
    return out_flat.reshape(NCLS, B).T
